# baseline (speedup 1.0000x reference)
import functools

import jax
import jax.numpy as jnp
from jax import lax
from jax.experimental import pallas as pl
from jax.experimental.pallas import tpu as pltpu
from jax.experimental.pallas import tpu_sc as plsc

NV = 10000
NF = 100000
B = 16
NVP = 10240
E2 = 3 * NF

NS = 16
EPT_F = 106 * 128
EPT_S = 43 * 128
F_TOTAL = NS * EPT_F
S_TOTAL = NS * EPT_S
E_PAD = F_TOTAL + S_TOTAL
PAD_COL = NV
SELF_COL = NV + 1
FAST_CORE = 1

BM = 200
N_BM = NV // BM
FBM = 400
N_FBM = NV // FBM


def _edge_block(a_v, b_v, idx_v, ones_v, n):
    @pl.loop(0, n // 16)
    def _compute(j):
        off = j * 16
        a = a_v[pl.ds(off, 16)]
        b = b_v[pl.ds(off, 16)]
        r = jnp.minimum(a, b)
        c = jnp.maximum(a, b)
        c = jnp.where(a == b, jnp.full((16,), SELF_COL, jnp.int32), c)
        idx_v[pl.ds(off, 16)] = r * NVP + c

    @pl.loop(0, n // 16)
    def _ones(j):
        ones_v[pl.ds(j * 16, 16)] = jnp.ones((16,), jnp.float32)


def _scatter_body(ra_hbm, rb_hbm, table_hbm,
                  a_v, b_v, idx_f, ones_f, idx_s, ones_s, sem):
    c = lax.axis_index("c")
    s = lax.axis_index("s")

    @pl.when(c == FAST_CORE)
    def _fast():
        base = s * EPT_F
        pltpu.sync_copy(ra_hbm.at[pl.ds(base, EPT_F)], a_v)
        pltpu.sync_copy(rb_hbm.at[pl.ds(base, EPT_F)], b_v)
        _edge_block(a_v, b_v, idx_f, ones_f, EPT_F)
        pltpu.make_async_copy(ones_f, table_hbm.at[idx_f], sem).start()
        pltpu.make_async_copy(ones_f, table_hbm.at[idx_f], sem).wait()

    @pl.when(c != FAST_CORE)
    def _slow():
        base = F_TOTAL + s * EPT_S
        pltpu.sync_copy(ra_hbm.at[pl.ds(base, EPT_S)],
                        a_v.at[pl.ds(0, EPT_S)])
        pltpu.sync_copy(rb_hbm.at[pl.ds(base, EPT_S)],
                        b_v.at[pl.ds(0, EPT_S)])
        _edge_block(a_v, b_v, idx_s, ones_s, EPT_S)
        pltpu.make_async_copy(ones_s, table_hbm.at[idx_s], sem).start()
        pltpu.make_async_copy(ones_s, table_hbm.at[idx_s], sem).wait()


@functools.cache
def _get_scatter_kernel():
    return pl.kernel(
        _scatter_body,
        out_type=(),
        mesh=plsc.VectorSubcoreMesh(core_axis_name="c", subcore_axis_name="s",
                                    num_cores=2, num_subcores=16),
        scratch_types=[
            pltpu.VMEM((EPT_F,), jnp.int32),
            pltpu.VMEM((EPT_F,), jnp.int32),
            pltpu.VMEM((EPT_F,), jnp.int32),
            pltpu.VMEM((EPT_F,), jnp.float32),
            pltpu.VMEM((EPT_S,), jnp.int32),
            pltpu.VMEM((EPT_S,), jnp.float32),
            pltpu.SemaphoreType.DMA,
        ],
    )


def _stripe_body(u_ref, xe_ref, xes_ref, out1_ref, tt_ref):
    m = pl.program_id(0)
    u = u_ref[...].reshape(BM, NVP)
    out1_ref[...] = jnp.dot(u, xe_ref[...], preferred_element_type=jnp.float32)

    @pl.when(m == 0)
    def _():
        tt_ref[...] = jnp.zeros_like(tt_ref)

    tt_ref[...] += lax.dot_general(u, xes_ref[...],
                                   (((0,), (0,)), ((), ())),
                                   preferred_element_type=jnp.float32)


_stripe_kernel = pl.pallas_call(
    _stripe_body,
    out_shape=(jax.ShapeDtypeStruct((NV, 64), jnp.float32),
               jax.ShapeDtypeStruct((NVP, 64), jnp.float32)),
    grid=(N_BM,),
    in_specs=[
        pl.BlockSpec((BM * NVP,), lambda m: (m,)),
        pl.BlockSpec((NVP, 64), lambda m: (0, 0)),
        pl.BlockSpec((BM, 64), lambda m: (m, 0)),
    ],
    out_specs=(pl.BlockSpec((BM, 64), lambda m: (m, 0)),
               pl.BlockSpec((NVP, 64), lambda m: (0, 0))),
)


def _reduce_body(o1_ref, tt_ref, xm_ref, out_ref):
    m = pl.program_id(0)
    s = o1_ref[...] + tt_ref[...]
    deg = s[:, 48:49]
    out = xm_ref[...] - s / deg
    p = jnp.reshape(jnp.sum(out * out), (1, 1))

    @pl.when(m == 0)
    def _():
        out_ref[...] = p

    @pl.when(m > 0)
    def _():
        out_ref[...] += p

    @pl.when(m == N_FBM - 1)
    def _():
        out_ref[...] = (out_ref[...] - float(NV)) / float(B)


_reduce_kernel = pl.pallas_call(
    _reduce_body,
    out_shape=jax.ShapeDtypeStruct((1, 1), jnp.float32),
    grid=(N_FBM,),
    in_specs=[
        pl.BlockSpec((FBM, 64), lambda m: (m, 0)),
        pl.BlockSpec((FBM, 64), lambda m: (m, 0)),
        pl.BlockSpec((FBM, 64), lambda m: (m, 0)),
    ],
    out_specs=pl.BlockSpec((1, 1), lambda m: (0, 0)),
)


def kernel(x, faces):
    f0 = faces[:, 0]
    f1 = faces[:, 1]
    f2 = faces[:, 2]
    ra = jnp.concatenate([f0, f1, f2])
    rb = jnp.concatenate([f1, f2, f0])
    pad = E_PAD - E2
    ra_p = jnp.concatenate([ra, jnp.zeros((pad,), jnp.int32)])
    rb_p = jnp.concatenate([rb, jnp.full((pad,), PAD_COL, jnp.int32)])

    table_ref = jax.new_ref(jnp.zeros((NV * NVP,), jnp.float32))
    _get_scatter_kernel()(ra_p, rb_p, table_ref)
    u_flat = table_ref[...]

    xt = x.transpose(1, 0, 2).reshape(NV, B * 3)
    xe = jnp.zeros((NVP, 64), jnp.float32)
    xe = xe.at[:NV, :48].set(xt)
    xe = xe.at[:NV, 48].set(1.0)
    xe = xe.at[SELF_COL, 48].set(1.0)
    xm = jnp.zeros((NV, 64), jnp.float32).at[:, :48].set(xt)

    out1, tt = _stripe_kernel(u_flat, xe, xe)
    loss = _reduce_kernel(out1, tt, xm)
    return loss[0, 0]


# speedup vs baseline: 1.9744x; 1.9744x over previous
import functools

import jax
import jax.numpy as jnp
from jax import lax
from jax.experimental import pallas as pl
from jax.experimental.pallas import tpu as pltpu
from jax.experimental.pallas import tpu_sc as plsc

NV = 10000
NF = 100000
B = 16
NVP = 10240
E2 = 3 * NF

NW = 32
EPT = 74 * 128
E_PAD = NW * EPT
PAD_COL = NV
SELF_COL = NV + 1

BM = 200
N_BM = NV // BM
FBM = 400
N_FBM = NV // FBM


def _edge_block(a_v, b_v, idx_v, ones_v, n):
    @pl.loop(0, n // 16)
    def _compute(j):
        off = j * 16
        a = a_v[pl.ds(off, 16)]
        b = b_v[pl.ds(off, 16)]
        r = jnp.minimum(a, b)
        c = jnp.maximum(a, b)
        c = jnp.where(a == b, jnp.full((16,), SELF_COL, jnp.int32), c)
        idx_v[pl.ds(off, 16)] = r * NVP + c

    @pl.loop(0, n // 16)
    def _ones(j):
        ones_v[pl.ds(j * 16, 16)] = jnp.ones((16,), jnp.float32)


def _scatter_body(ra_hbm, rb_hbm, table_hbm, a_v, b_v, idx_v, ones_v, sem):
    wid = lax.axis_index("s") * 2 + lax.axis_index("c")
    base = wid * EPT
    pltpu.sync_copy(ra_hbm.at[pl.ds(base, EPT)], a_v)
    pltpu.sync_copy(rb_hbm.at[pl.ds(base, EPT)], b_v)
    _edge_block(a_v, b_v, idx_v, ones_v, EPT)
    pltpu.make_async_copy(ones_v, table_hbm.at[idx_v], sem).start()
    pltpu.make_async_copy(ones_v, table_hbm.at[idx_v], sem).wait()


@functools.cache
def _get_scatter_kernel():
    return pl.kernel(
        _scatter_body,
        out_type=(),
        mesh=plsc.VectorSubcoreMesh(core_axis_name="c", subcore_axis_name="s",
                                    num_cores=2, num_subcores=16),
        scratch_types=[
            pltpu.VMEM((EPT,), jnp.int32),
            pltpu.VMEM((EPT,), jnp.int32),
            pltpu.VMEM((EPT,), jnp.int32),
            pltpu.VMEM((EPT,), jnp.float32),
            pltpu.SemaphoreType.DMA,
        ],
    )


def _stripe_body(u_ref, xe_ref, xes_ref, out1_ref, tt_ref):
    m = pl.program_id(0)
    u = u_ref[...].reshape(BM, NVP)
    out1_ref[...] = jnp.dot(u, xe_ref[...], preferred_element_type=jnp.float32)

    @pl.when(m == 0)
    def _():
        tt_ref[...] = jnp.zeros_like(tt_ref)

    tt_ref[...] += lax.dot_general(u, xes_ref[...],
                                   (((0,), (0,)), ((), ())),
                                   preferred_element_type=jnp.float32)


_stripe_kernel = pl.pallas_call(
    _stripe_body,
    out_shape=(jax.ShapeDtypeStruct((NV, 64), jnp.float32),
               jax.ShapeDtypeStruct((NVP, 64), jnp.float32)),
    grid=(N_BM,),
    in_specs=[
        pl.BlockSpec((BM * NVP,), lambda m: (m,)),
        pl.BlockSpec((NVP, 64), lambda m: (0, 0)),
        pl.BlockSpec((BM, 64), lambda m: (m, 0)),
    ],
    out_specs=(pl.BlockSpec((BM, 64), lambda m: (m, 0)),
               pl.BlockSpec((NVP, 64), lambda m: (0, 0))),
)


def _reduce_body(o1_ref, tt_ref, xm_ref, out_ref):
    m = pl.program_id(0)
    s = o1_ref[...] + tt_ref[...]
    deg = s[:, 48:49]
    out = xm_ref[...] - s / deg
    p = jnp.reshape(jnp.sum(out * out), (1, 1))

    @pl.when(m == 0)
    def _():
        out_ref[...] = p

    @pl.when(m > 0)
    def _():
        out_ref[...] += p

    @pl.when(m == N_FBM - 1)
    def _():
        out_ref[...] = (out_ref[...] - float(NV)) / float(B)


_reduce_kernel = pl.pallas_call(
    _reduce_body,
    out_shape=jax.ShapeDtypeStruct((1, 1), jnp.float32),
    grid=(N_FBM,),
    in_specs=[
        pl.BlockSpec((FBM, 64), lambda m: (m, 0)),
        pl.BlockSpec((FBM, 64), lambda m: (m, 0)),
        pl.BlockSpec((FBM, 64), lambda m: (m, 0)),
    ],
    out_specs=pl.BlockSpec((1, 1), lambda m: (0, 0)),
)


def kernel(x, faces):
    f0 = faces[:, 0]
    f1 = faces[:, 1]
    f2 = faces[:, 2]
    ra = jnp.concatenate([f0, f1, f2])
    rb = jnp.concatenate([f1, f2, f0])
    pad = E_PAD - E2
    ra_p = jnp.concatenate([ra, jnp.arange(pad, dtype=jnp.int32)])
    rb_p = jnp.concatenate([rb, jnp.full((pad,), PAD_COL, jnp.int32)])

    table_ref = jax.new_ref(jnp.zeros((NV * NVP,), jnp.float32))
    _get_scatter_kernel()(ra_p, rb_p, table_ref)
    u_flat = table_ref[...]

    xt = x.transpose(1, 0, 2).reshape(NV, B * 3)
    xe = jnp.zeros((NVP, 64), jnp.float32)
    xe = xe.at[:NV, :48].set(xt)
    xe = xe.at[:NV, 48].set(1.0)
    xe = xe.at[SELF_COL, 48].set(1.0)
    xm = jnp.zeros((NV, 64), jnp.float32).at[:, :48].set(xt)

    out1, tt = _stripe_kernel(u_flat, xe, xe)
    loss = _reduce_kernel(out1, tt, xm)
    return loss[0, 0]


# speedup vs baseline: 2.2144x; 1.1216x over previous
import functools

import jax
import jax.numpy as jnp
from jax import lax
from jax.experimental import pallas as pl
from jax.experimental.pallas import tpu as pltpu
from jax.experimental.pallas import tpu_sc as plsc

NV = 10000
NF = 100000
B = 16
NVP = 10240
E2 = 3 * NF

NW = 32
EPT = 74 * 128
E_PAD = NW * EPT
PAD_COL = NV
SELF_COL = NV + 1

BM = 200
N_BM = NV // BM
FBM = 400
N_FBM = NV // FBM


def _edge_block(a_v, b_v, idx_v, ones_v, n):
    @pl.loop(0, n // 16)
    def _compute(j):
        off = j * 16
        a = a_v[pl.ds(off, 16)]
        b = b_v[pl.ds(off, 16)]
        r = jnp.minimum(a, b)
        c = jnp.maximum(a, b)
        c = jnp.where(a == b, jnp.full((16,), SELF_COL, jnp.int32), c)
        idx_v[pl.ds(off, 16)] = r * NVP + c

    @pl.loop(0, n // 16)
    def _ones(j):
        ones_v[pl.ds(j * 16, 16)] = jnp.ones((16,), jnp.float32)


def _scatter_body(ra_hbm, rb_hbm, table_hbm, a_v, b_v, idx_v, ones_v, sem):
    wid = lax.axis_index("s") * 2 + lax.axis_index("c")
    base = wid * EPT
    pltpu.sync_copy(ra_hbm.at[pl.ds(base, EPT)], a_v)
    pltpu.sync_copy(rb_hbm.at[pl.ds(base, EPT)], b_v)
    _edge_block(a_v, b_v, idx_v, ones_v, EPT)
    pltpu.make_async_copy(ones_v, table_hbm.at[idx_v], sem).start()
    pltpu.make_async_copy(ones_v, table_hbm.at[idx_v], sem).wait()


@functools.cache
def _get_scatter_kernel():
    return pl.kernel(
        _scatter_body,
        out_type=(),
        mesh=plsc.VectorSubcoreMesh(core_axis_name="c", subcore_axis_name="s",
                                    num_cores=2, num_subcores=16),
        scratch_types=[
            pltpu.VMEM((EPT,), jnp.int32),
            pltpu.VMEM((EPT,), jnp.int32),
            pltpu.VMEM((EPT,), jnp.int32),
            pltpu.VMEM((EPT,), jnp.float32),
            pltpu.SemaphoreType.DMA,
        ],
    )


def _stripe_body(u_ref, xe_ref, xes_ref, out1_ref, tt_ref):
    m = pl.program_id(0)
    u = u_ref[...].reshape(BM, NVP).astype(jnp.bfloat16)
    out1_ref[...] = jnp.dot(u, xe_ref[...], preferred_element_type=jnp.float32)

    @pl.when(m == 0)
    def _():
        tt_ref[...] = jnp.zeros_like(tt_ref)

    tt_ref[...] += lax.dot_general(u, xes_ref[...],
                                   (((0,), (0,)), ((), ())),
                                   preferred_element_type=jnp.float32)


_stripe_kernel = pl.pallas_call(
    _stripe_body,
    out_shape=(jax.ShapeDtypeStruct((NV, 64), jnp.float32),
               jax.ShapeDtypeStruct((NVP, 64), jnp.float32)),
    grid=(N_BM,),
    in_specs=[
        pl.BlockSpec((BM * NVP,), lambda m: (m,)),
        pl.BlockSpec((NVP, 64), lambda m: (0, 0)),
        pl.BlockSpec((BM, 64), lambda m: (m, 0)),
    ],
    out_specs=(pl.BlockSpec((BM, 64), lambda m: (m, 0)),
               pl.BlockSpec((NVP, 64), lambda m: (0, 0))),
)


def _reduce_body(o1_ref, tt_ref, xm_ref, out_ref):
    m = pl.program_id(0)
    s = o1_ref[...] + tt_ref[...]
    deg = s[:, 48:49]
    out = xm_ref[...] - s / deg
    p = jnp.reshape(jnp.sum(out * out), (1, 1))

    @pl.when(m == 0)
    def _():
        out_ref[...] = p

    @pl.when(m > 0)
    def _():
        out_ref[...] += p

    @pl.when(m == N_FBM - 1)
    def _():
        out_ref[...] = (out_ref[...] - float(NV)) / float(B)


_reduce_kernel = pl.pallas_call(
    _reduce_body,
    out_shape=jax.ShapeDtypeStruct((1, 1), jnp.float32),
    grid=(N_FBM,),
    in_specs=[
        pl.BlockSpec((FBM, 64), lambda m: (m, 0)),
        pl.BlockSpec((FBM, 64), lambda m: (m, 0)),
        pl.BlockSpec((FBM, 64), lambda m: (m, 0)),
    ],
    out_specs=pl.BlockSpec((1, 1), lambda m: (0, 0)),
)


def kernel(x, faces):
    f0 = faces[:, 0]
    f1 = faces[:, 1]
    f2 = faces[:, 2]
    ra = jnp.concatenate([f0, f1, f2])
    rb = jnp.concatenate([f1, f2, f0])
    pad = E_PAD - E2
    ra_p = jnp.concatenate([ra, jnp.arange(pad, dtype=jnp.int32)])
    rb_p = jnp.concatenate([rb, jnp.full((pad,), PAD_COL, jnp.int32)])

    table_ref = jax.new_ref(jnp.zeros((NV * NVP,), jnp.float32))
    _get_scatter_kernel()(ra_p, rb_p, table_ref)
    u_flat = table_ref[...]

    xt = x.transpose(1, 0, 2).reshape(NV, B * 3)
    xe = jnp.zeros((NVP, 64), jnp.float32)
    xe = xe.at[:NV, :48].set(xt)
    xe = xe.at[:NV, 48].set(1.0)
    xe = xe.at[SELF_COL, 48].set(1.0)
    xe16 = xe.astype(jnp.bfloat16)
    xm = jnp.zeros((NV, 64), jnp.float32).at[:, :48].set(xt)

    out1, tt = _stripe_kernel(u_flat, xe16, xe16)
    loss = _reduce_kernel(out1, tt, xm)
    return loss[0, 0]
